# baseline (speedup 1.0000x reference)
import functools
import math

import jax
import jax.numpy as jnp
from jax import lax
from jax.experimental import pallas as pl
from jax.experimental.pallas import tpu as pltpu
from jax.experimental.pallas import tpu_sc as plsc

_LANES = 16
_IDX_W = 128
_RING = 3


def _embed_sc(tokens_2d, table_slots, scale):
    n_rows, idx_w = tokens_2d.shape
    vocab, slot_w = table_slots.shape
    dim = slot_w // 2
    info = plsc.get_sparse_core_info()
    n_workers = info.num_cores * info.num_subcores
    rows_per_w = n_rows // n_workers
    total = n_rows * idx_w
    n_full = (rows_per_w // _RING) * _RING

    mesh = plsc.VectorSubcoreMesh(core_axis_name="c", subcore_axis_name="s")

    @functools.partial(
        pl.kernel,
        mesh=mesh,
        out_type=jax.ShapeDtypeStruct((total, dim), jnp.float32),
        scratch_types=[
            pltpu.VMEM((rows_per_w, idx_w), jnp.int32),
            [pltpu.VMEM((idx_w, slot_w), jnp.float32) for _ in range(_RING)],
            [pltpu.VMEM((idx_w, dim), jnp.float32) for _ in range(_RING)],
            [pltpu.SemaphoreType.DMA for _ in range(_RING)],
            [pltpu.SemaphoreType.DMA for _ in range(_RING)],
        ],
        compiler_params=pltpu.CompilerParams(use_tc_tiling_on_sc=True),
    )
    def k(tok_hbm, tab_hbm, out_hbm, idx_v, bufs, obufs, gsems, osems):
        w = lax.axis_index("s") * info.num_cores + lax.axis_index("c")
        pltpu.sync_copy(tok_hbm.at[pl.ds(w * rows_per_w, rows_per_w)], idx_v)
        tbase = w * rows_per_w * idx_w

        def wait_gather(s):
            pltpu.make_async_copy(tab_hbm.at[idx_v.at[0]], bufs[s], gsems[s]).wait()

        def wait_out(s):
            pltpu.make_async_copy(
                out_hbm.at[pl.ds(0, idx_w)], obufs[s], osems[s]
            ).wait()

        def compact(s):
            def row_body(rr, _):
                for u in range(4):
                    r = 4 * rr + u
                    for k16 in range(dim // _LANES):
                        sl = pl.ds(k16 * _LANES, _LANES)
                        obufs[s][r, sl] = bufs[s][r, sl] * scale
                return 0

            lax.fori_loop(0, idx_w // 4, row_body, 0)

        def fire_out(s, j):
            pltpu.async_copy(
                obufs[s], out_hbm.at[pl.ds(tbase + j * idx_w, idx_w)], osems[s]
            )

        for s in range(_RING):
            pltpu.async_copy(tab_hbm.at[idx_v.at[s]], bufs[s], gsems[s])

        def body(m, _):
            for s in range(_RING):
                j = _RING * m + s
                wait_gather(s)

                @pl.when(m > 0)
                def _():
                    wait_out(s)

                compact(s)
                fire_out(s, j)

                @pl.when(j + _RING < rows_per_w)
                def _():
                    pltpu.async_copy(
                        tab_hbm.at[idx_v.at[j + _RING]], bufs[s], gsems[s]
                    )

            return 0

        lax.fori_loop(0, rows_per_w // _RING, body, 0)
        for t in range(n_full, rows_per_w):
            s = t % _RING
            wait_gather(s)
            wait_out(s)
            compact(s)
            fire_out(s, t)
        for s in range(_RING):
            wait_out(s)

    return k(tokens_2d, table_slots)


def kernel(tokens, embedding_weight):
    b0, b1 = tokens.shape
    vocab, dim = embedding_weight.shape
    scale = math.sqrt(dim)
    toks = tokens.reshape(b0 * b1 // _IDX_W, _IDX_W)
    table_slots = jnp.pad(embedding_weight, ((0, 0), (0, dim)))
    out = _embed_sc(toks, table_slots, scale)
    return out.reshape(b0, b1, dim)


# speedup vs baseline: 1.0009x; 1.0009x over previous
import functools
import math

import jax
import jax.numpy as jnp
from jax import lax
from jax.experimental import pallas as pl
from jax.experimental.pallas import tpu as pltpu
from jax.experimental.pallas import tpu_sc as plsc

_LANES = 16
_IDX_W = 128
_RING = 4
_ORING = 2


def _embed_sc(tokens_2d, table_slots, scale):
    n_rows, idx_w = tokens_2d.shape
    vocab, slot_w = table_slots.shape
    dim = slot_w // 2
    info = plsc.get_sparse_core_info()
    n_workers = info.num_cores * info.num_subcores
    rows_per_w = n_rows // n_workers
    total = n_rows * idx_w

    mesh = plsc.VectorSubcoreMesh(core_axis_name="c", subcore_axis_name="s")

    @functools.partial(
        pl.kernel,
        mesh=mesh,
        out_type=jax.ShapeDtypeStruct((total, dim), jnp.float32),
        scratch_types=[
            pltpu.VMEM((rows_per_w, idx_w), jnp.int32),
            [pltpu.VMEM((idx_w, slot_w), jnp.float32) for _ in range(_RING)],
            [pltpu.VMEM((idx_w, dim), jnp.float32) for _ in range(_ORING)],
            [pltpu.SemaphoreType.DMA for _ in range(_RING)],
            [pltpu.SemaphoreType.DMA for _ in range(_ORING)],
        ],
        compiler_params=pltpu.CompilerParams(use_tc_tiling_on_sc=True),
    )
    def k(tok_hbm, tab_hbm, out_hbm, idx_v, bufs, obufs, gsems, osems):
        w = lax.axis_index("s") * info.num_cores + lax.axis_index("c")
        pltpu.sync_copy(tok_hbm.at[pl.ds(w * rows_per_w, rows_per_w)], idx_v)
        tbase = w * rows_per_w * idx_w

        def wait_gather(s):
            pltpu.make_async_copy(tab_hbm.at[idx_v.at[0]], bufs[s], gsems[s]).wait()

        def wait_out(so):
            pltpu.make_async_copy(
                out_hbm.at[pl.ds(0, idx_w)], obufs[so], osems[so]
            ).wait()

        def compact(s, so):
            def row_body(rr, _):
                for u in range(4):
                    r = 4 * rr + u
                    for k16 in range(dim // _LANES):
                        sl = pl.ds(k16 * _LANES, _LANES)
                        obufs[so][r, sl] = bufs[s][r, sl] * scale
                return 0

            lax.fori_loop(0, idx_w // 4, row_body, 0)

        def fire_out(so, j):
            pltpu.async_copy(
                obufs[so], out_hbm.at[pl.ds(tbase + j * idx_w, idx_w)], osems[so]
            )

        for s in range(_RING):
            pltpu.async_copy(tab_hbm.at[idx_v.at[s]], bufs[s], gsems[s])

        def body(m, _):
            for s in range(_RING):
                j = _RING * m + s
                so = s % _ORING
                wait_gather(s)
                if s < _ORING:
                    @pl.when(m > 0)
                    def _():
                        wait_out(so)

                else:
                    wait_out(so)
                compact(s, so)
                fire_out(so, j)

                @pl.when(j + _RING < rows_per_w)
                def _():
                    pltpu.async_copy(
                        tab_hbm.at[idx_v.at[j + _RING]], bufs[s], gsems[s]
                    )

            return 0

        lax.fori_loop(0, rows_per_w // _RING, body, 0)
        for so in range(_ORING):
            wait_out(so)

    return k(tokens_2d, table_slots)


def kernel(tokens, embedding_weight):
    b0, b1 = tokens.shape
    vocab, dim = embedding_weight.shape
    scale = math.sqrt(dim)
    toks = tokens.reshape(b0 * b1 // _IDX_W, _IDX_W)
    table_slots = jnp.pad(embedding_weight, ((0, 0), (0, dim)))
    out = _embed_sc(toks, table_slots, scale)
    return out.reshape(b0, b1, dim)
